# baseline (speedup 1.0000x reference)
import jax
import jax.numpy as jnp
from jax.experimental import pallas as pl
from jax.experimental.pallas import tpu as pltpu

_ROW_BLOCK = 256


def _emb_kernel(x_ref, w_ref, cls_ref, out_ref):
    x = x_ref[...]
    xc = jnp.where(jnp.isnan(x), jnp.float32(0.0), x)
    w = w_ref[...]
    y = xc[:, :, None] * w[None, :, :]
    out_ref[:, : w.shape[0], :] = y
    out_ref[:, w.shape[0] :, :] = jnp.broadcast_to(
        cls_ref[...][None], (x.shape[0], 1, w.shape[1])
    )


def kernel(x, emb_weight, cls_token):
    b, L = x.shape
    D = emb_weight.shape[1]
    w = emb_weight[1 : L + 1]
    cls = cls_token.reshape(1, D)
    rb = _ROW_BLOCK
    grid = (b // rb,)
    return pl.pallas_call(
        _emb_kernel,
        grid=grid,
        in_specs=[
            pl.BlockSpec((rb, L), lambda i: (i, 0)),
            pl.BlockSpec((L, D), lambda i: (0, 0)),
            pl.BlockSpec((1, D), lambda i: (0, 0)),
        ],
        out_specs=pl.BlockSpec((rb, L + 1, D), lambda i: (i, 0, 0)),
        out_shape=jax.ShapeDtypeStruct((b, L + 1, D), jnp.float32),
        compiler_params=pltpu.CompilerParams(
            dimension_semantics=("parallel",),
        ),
    )(x, w, cls)


# speedup vs baseline: 1.1410x; 1.1410x over previous
import jax
import jax.numpy as jnp
from jax.experimental import pallas as pl
from jax.experimental.pallas import tpu as pltpu

_ROW_BLOCK = 256


def _emb_kernel(x_ref, w_ref, cls_ref, out_ref):
    rb, L = x_ref.shape
    D = w_ref.shape[1]
    x = x_ref[...]
    xc = jnp.where(jnp.isnan(x), jnp.float32(0.0), x)
    w = w_ref[...]
    y = xc[:, :, None] * w[None, :, :]
    out_ref[:, : L * D] = y.reshape(rb, L * D)
    out_ref[:, L * D :] = jnp.broadcast_to(cls_ref[...], (rb, D))


def kernel(x, emb_weight, cls_token):
    b, L = x.shape
    D = emb_weight.shape[1]
    w = emb_weight[1 : L + 1]
    cls = cls_token.reshape(1, D)
    rb = _ROW_BLOCK
    out2d = pl.pallas_call(
        _emb_kernel,
        grid=(b // rb,),
        in_specs=[
            pl.BlockSpec((rb, L), lambda i: (i, 0)),
            pl.BlockSpec((L, D), lambda i: (0, 0)),
            pl.BlockSpec((1, D), lambda i: (0, 0)),
        ],
        out_specs=pl.BlockSpec((rb, (L + 1) * D), lambda i: (i, 0)),
        out_shape=jax.ShapeDtypeStruct((b, (L + 1) * D), jnp.float32),
        compiler_params=pltpu.CompilerParams(
            dimension_semantics=("parallel",),
        ),
    )(x, w, cls)
    return out2d.reshape(b, L + 1, D)
